# baseline (speedup 1.0000x reference)
import jax
import jax.numpy as jnp
from jax.experimental import pallas as pl
from jax.experimental.pallas import tpu as pltpu

_G = 512
_BRA = 640
_BRB = 3200
_K = 128


def _pass_a(lo_ref, hi_ref, x_ref, bc_ref, br_ref, w1_ref, w2_ref, y_ref,
            sum_ref, max_ref, bbc_ref):
    i = pl.program_id(0)

    @pl.when(i == 0)
    def _init():
        sum_ref[...] = jnp.zeros_like(sum_ref)
        max_ref[...] = jnp.full_like(max_ref, -jnp.inf)

    b = bc_ref[0, :, :]
    brow = br_ref[0, :, :]
    x = x_ref[...]
    s_lo = lo_ref[i]
    s_hi = hi_ref[i]
    wlo = (s_lo // 8) * 8

    bbc_ref[...] = jnp.broadcast_to(b, (_BRA, 128))

    def body(s, carry):
        m = bbc_ref[...] == s
        xv = x_ref[...]
        sm = jnp.sum(jnp.where(m, xv, 0.0), axis=0, keepdims=True)
        mx = jnp.max(jnp.where(m, xv, -jnp.inf), axis=0, keepdims=True)
        sum_ref[pl.ds(s, 1), :] = sum_ref[pl.ds(s, 1), :] + sm
        max_ref[pl.ds(s, 1), :] = jnp.maximum(max_ref[pl.ds(s, 1), :], mx)
        return carry

    jax.lax.fori_loop(s_lo, s_hi + 1, body, 0)

    @pl.when(i == pl.num_programs(0) - 1)
    def _finish():
        mx = max_ref[0:_G, :]
        mx = jnp.where(mx == -jnp.inf, 0.0, mx)
        sm = sum_ref[0:_G, :]
        w1 = w1_ref[...]
        w2 = w2_ref[...]
        h1 = jnp.maximum(jnp.dot(mx, w1, preferred_element_type=jnp.float32), 0.0)
        o1 = jnp.dot(h1, w2, preferred_element_type=jnp.float32)
        h2 = jnp.maximum(jnp.dot(sm, w1, preferred_element_type=jnp.float32), 0.0)
        o2 = jnp.dot(h2, w2, preferred_element_type=jnp.float32)
        y_ref[...] = jnp.maximum(o1 + o2, 0.0)


def _pass_b(lo_ref, hi_ref, x_ref, bc_ref, y_ref, o_ref):
    i = pl.program_id(0)
    b = bc_ref[0, :, :]
    s_lo = lo_ref[i]
    s_hi = hi_ref[i]

    wlo = (s_lo // 8) * 8
    kio = jax.lax.broadcasted_iota(jnp.int32, (_BRB, _K), 1)
    m = (kio == (b - wlo)).astype(jnp.float32)
    ys = y_ref[pl.ds(wlo, _K), :]
    rows = jax.lax.dot_general(m, ys, (((1,), (0,)), ((), ())),
                               preferred_element_type=jnp.float32)
    o_ref[...] = rows

    def body(s, carry):
        yy = y_ref[pl.ds(s, 1), :]
        mm = b == s
        o_ref[...] = jnp.where(mm, yy, o_ref[...])
        return carry

    jax.lax.fori_loop(wlo + _K, s_hi + 1, body, 0)
    o_ref[...] = o_ref[...] * x_ref[...]


def kernel(x, batch, W1, W2):
    n, c = x.shape
    nba = n // _BRA
    bcola = batch.reshape(nba, _BRA, 1)
    browa = batch.reshape(nba, 1, _BRA)
    bloa = bcola[:, 0, 0]
    bhia = bcola[:, _BRA - 1, 0]

    y = pl.pallas_call(
        _pass_a,
        grid=(nba,),
        in_specs=[
            pl.BlockSpec(memory_space=pltpu.SMEM),
            pl.BlockSpec(memory_space=pltpu.SMEM),
            pl.BlockSpec((_BRA, c), lambda i: (i, 0)),
            pl.BlockSpec((1, _BRA, 1), lambda i: (i, 0, 0)),
            pl.BlockSpec((1, 1, _BRA), lambda i: (i, 0, 0)),
            pl.BlockSpec((c, c // 8), lambda i: (0, 0)),
            pl.BlockSpec((c // 8, c), lambda i: (0, 0)),
        ],
        out_specs=pl.BlockSpec((_G, c), lambda i: (0, 0)),
        out_shape=jax.ShapeDtypeStruct((_G, c), jnp.float32),
        scratch_shapes=[
            pltpu.VMEM((_G + _K, c), jnp.float32),
            pltpu.VMEM((_G, c), jnp.float32),
            pltpu.VMEM((_BRA, c), jnp.int32),
        ],
        compiler_params=pltpu.CompilerParams(
            dimension_semantics=("arbitrary",),
        ),
    )(bloa, bhia, x, bcola, browa, W1, W2)

    ypad = jnp.concatenate([y, jnp.zeros((_K, c), jnp.float32)], axis=0)

    nbb = n // _BRB
    bcolb = batch.reshape(nbb, _BRB, 1)
    blob = bcolb[:, 0, 0]
    bhib = bcolb[:, _BRB - 1, 0]

    out = pl.pallas_call(
        _pass_b,
        grid=(nbb,),
        in_specs=[
            pl.BlockSpec(memory_space=pltpu.SMEM),
            pl.BlockSpec(memory_space=pltpu.SMEM),
            pl.BlockSpec((_BRB, c), lambda i: (i, 0)),
            pl.BlockSpec((1, _BRB, 1), lambda i: (i, 0, 0)),
            pl.BlockSpec((_G + _K, c), lambda i: (0, 0)),
        ],
        out_specs=pl.BlockSpec((_BRB, c), lambda i: (i, 0)),
        out_shape=jax.ShapeDtypeStruct((n, c), jnp.float32),
        compiler_params=pltpu.CompilerParams(
            dimension_semantics=("arbitrary",),
        ),
    )(blob, bhib, x, bcolb, ypad)
    return out


# speedup vs baseline: 1.2035x; 1.2035x over previous
import jax
import jax.numpy as jnp
from jax.experimental import pallas as pl
from jax.experimental.pallas import tpu as pltpu

_G = 512
_BRA = 1280
_BRB = 6400
_K = 128


def _pass_a(lo_ref, hi_ref, x_ref, bc_ref, br_ref, w1_ref, w2_ref, y_ref,
            sum_ref, max_ref, bbc_ref):
    i = pl.program_id(0)

    @pl.when(i == 0)
    def _init():
        sum_ref[...] = jnp.zeros_like(sum_ref)
        max_ref[...] = jnp.full_like(max_ref, -jnp.inf)

    b = bc_ref[0, :, :]
    brow = br_ref[0, :, :]
    x = x_ref[...]
    s_lo = lo_ref[i]
    s_hi = hi_ref[i]
    wlo = (s_lo // 8) * 8

    bbc_ref[...] = jnp.broadcast_to(b, (_BRA, 128))

    def body(s, carry):
        m = bbc_ref[...] == s
        xv = x_ref[...]
        sm = jnp.sum(jnp.where(m, xv, 0.0), axis=0, keepdims=True)
        mx = jnp.max(jnp.where(m, xv, -jnp.inf), axis=0, keepdims=True)
        sum_ref[pl.ds(s, 1), :] = sum_ref[pl.ds(s, 1), :] + sm
        max_ref[pl.ds(s, 1), :] = jnp.maximum(max_ref[pl.ds(s, 1), :], mx)
        return carry

    jax.lax.fori_loop(s_lo, s_hi + 1, body, 0)

    @pl.when(i == pl.num_programs(0) - 1)
    def _finish():
        mx = max_ref[0:_G, :]
        mx = jnp.where(mx == -jnp.inf, 0.0, mx)
        sm = sum_ref[0:_G, :]
        w1 = w1_ref[...]
        w2 = w2_ref[...]
        h1 = jnp.maximum(jnp.dot(mx, w1, preferred_element_type=jnp.float32), 0.0)
        o1 = jnp.dot(h1, w2, preferred_element_type=jnp.float32)
        h2 = jnp.maximum(jnp.dot(sm, w1, preferred_element_type=jnp.float32), 0.0)
        o2 = jnp.dot(h2, w2, preferred_element_type=jnp.float32)
        y_ref[...] = jnp.maximum(o1 + o2, 0.0)


def _pass_b(lo_ref, hi_ref, x_ref, bc_ref, y_ref, o_ref):
    i = pl.program_id(0)
    b = bc_ref[0, :, :]
    s_lo = lo_ref[i]
    s_hi = hi_ref[i]

    wlo = (s_lo // 8) * 8
    kio = jax.lax.broadcasted_iota(jnp.int32, (_BRB, _K), 1)
    m = (kio == (b - wlo)).astype(jnp.float32)
    ys = y_ref[pl.ds(wlo, _K), :]
    rows = jax.lax.dot_general(m, ys, (((1,), (0,)), ((), ())),
                               preferred_element_type=jnp.float32)
    o_ref[...] = rows

    def body(s, carry):
        yy = y_ref[pl.ds(s, 1), :]
        mm = b == s
        o_ref[...] = jnp.where(mm, yy, o_ref[...])
        return carry

    jax.lax.fori_loop(wlo + _K, s_hi + 1, body, 0)
    o_ref[...] = o_ref[...] * x_ref[...]


def kernel(x, batch, W1, W2):
    n, c = x.shape
    nba = n // _BRA
    bcola = batch.reshape(nba, _BRA, 1)
    browa = batch.reshape(nba, 1, _BRA)
    bloa = bcola[:, 0, 0]
    bhia = bcola[:, _BRA - 1, 0]

    y = pl.pallas_call(
        _pass_a,
        grid=(nba,),
        in_specs=[
            pl.BlockSpec(memory_space=pltpu.SMEM),
            pl.BlockSpec(memory_space=pltpu.SMEM),
            pl.BlockSpec((_BRA, c), lambda i: (i, 0)),
            pl.BlockSpec((1, _BRA, 1), lambda i: (i, 0, 0)),
            pl.BlockSpec((1, 1, _BRA), lambda i: (i, 0, 0)),
            pl.BlockSpec((c, c // 8), lambda i: (0, 0)),
            pl.BlockSpec((c // 8, c), lambda i: (0, 0)),
        ],
        out_specs=pl.BlockSpec((_G, c), lambda i: (0, 0)),
        out_shape=jax.ShapeDtypeStruct((_G, c), jnp.float32),
        scratch_shapes=[
            pltpu.VMEM((_G + _K, c), jnp.float32),
            pltpu.VMEM((_G, c), jnp.float32),
            pltpu.VMEM((_BRA, c), jnp.int32),
        ],
        compiler_params=pltpu.CompilerParams(
            dimension_semantics=("arbitrary",),
        ),
    )(bloa, bhia, x, bcola, browa, W1, W2)

    ypad = jnp.concatenate([y, jnp.zeros((_K, c), jnp.float32)], axis=0)

    nbb = n // _BRB
    bcolb = batch.reshape(nbb, _BRB, 1)
    blob = bcolb[:, 0, 0]
    bhib = bcolb[:, _BRB - 1, 0]

    out = pl.pallas_call(
        _pass_b,
        grid=(nbb,),
        in_specs=[
            pl.BlockSpec(memory_space=pltpu.SMEM),
            pl.BlockSpec(memory_space=pltpu.SMEM),
            pl.BlockSpec((_BRB, c), lambda i: (i, 0)),
            pl.BlockSpec((1, _BRB, 1), lambda i: (i, 0, 0)),
            pl.BlockSpec((_G + _K, c), lambda i: (0, 0)),
        ],
        out_specs=pl.BlockSpec((_BRB, c), lambda i: (i, 0)),
        out_shape=jax.ShapeDtypeStruct((n, c), jnp.float32),
        compiler_params=pltpu.CompilerParams(
            dimension_semantics=("arbitrary",),
        ),
    )(blob, bhib, x, bcolb, ypad)
    return out
